# Initial kernel scaffold; baseline (speedup 1.0000x reference)
import jax
import jax.numpy as jnp
from jax.experimental import pallas as pl


def kernel(X, table):
    raise NotImplementedError("write your pallas kernel here")


# speedup vs baseline: 1.4591x; 1.4591x over previous
import functools

import jax
import jax.numpy as jnp
from jax import lax
from jax.experimental import pallas as pl
from jax.experimental.pallas import tpu as pltpu
from jax.experimental.pallas import tpu_sc as plsc

D = 32
NC, NS = 2, 16
NW = NC * NS
IW = 128
K = 8
CHUNK = K * IW


def _make_gather(B):
    bpw = B // NW
    nchunks = bpw // CHUNK
    irows_pw = bpw // IW

    mesh = plsc.VectorSubcoreMesh(core_axis_name="c", subcore_axis_name="s")

    @functools.partial(
        pl.kernel,
        out_type=jax.ShapeDtypeStruct((B, D), jnp.float32),
        mesh=mesh,
        scratch_types=[
            pltpu.VMEM((K, IW), jnp.int32),
            pltpu.VMEM((CHUNK, D), jnp.float32),
            pltpu.SemaphoreType.DMA,
        ],
        compiler_params=pltpu.CompilerParams(use_tc_tiling_on_sc=False),
    )
    def gather(table_hbm, idx_hbm, out_hbm, idx_v, rows_v, sem):
        wid = lax.axis_index("s") * NC + lax.axis_index("c")
        irow0 = wid * irows_pw
        obase = wid * bpw

        @pl.loop(0, nchunks)
        def _(c):
            pltpu.sync_copy(idx_hbm.at[pl.ds(irow0 + c * K, K)], idx_v)
            copies = [
                pltpu.async_copy(
                    table_hbm.at[idx_v.at[j]],
                    rows_v.at[pl.ds(j * IW, IW)],
                    sem,
                )
                for j in range(K)
            ]
            for cp in copies:
                cp.wait()
            pltpu.sync_copy(rows_v, out_hbm.at[pl.ds(obase + c * CHUNK, CHUNK)])

    return gather


def kernel(X, table):
    Bb, T = X.shape
    B = Bb * T
    idx = X.reshape(B // IW, IW).astype(jnp.int32)
    out = _make_gather(B)(table, idx)
    return out.reshape(Bb, T, D)


# speedup vs baseline: 1.4921x; 1.0226x over previous
import functools

import jax
import jax.numpy as jnp
from jax import lax
from jax.experimental import pallas as pl
from jax.experimental.pallas import tpu as pltpu
from jax.experimental.pallas import tpu_sc as plsc

D = 32
NC, NS = 2, 16
NW = NC * NS
IW = 128
K = 8
CHUNK = K * IW


def _make_gather(B):
    bpw = B // NW
    nchunks = bpw // CHUNK
    irows_pw = bpw // IW
    assert nchunks % 2 == 1 and nchunks >= 5

    mesh = plsc.VectorSubcoreMesh(core_axis_name="c", subcore_axis_name="s")

    @functools.partial(
        pl.kernel,
        out_type=jax.ShapeDtypeStruct((B, D), jnp.float32),
        mesh=mesh,
        scratch_types=[
            pltpu.VMEM((2, K, IW), jnp.int32),
            pltpu.VMEM((2, CHUNK, D), jnp.float32),
            pltpu.SemaphoreType.DMA,
            pltpu.SemaphoreType.DMA,
            pltpu.SemaphoreType.DMA,
            pltpu.SemaphoreType.DMA,
            pltpu.SemaphoreType.DMA,
        ],
        compiler_params=pltpu.CompilerParams(use_tc_tiling_on_sc=False),
    )
    def gather(table_hbm, idx_hbm, out_hbm, idx_v, rows_v,
               sem_i0, sem_i1, sem_o0, sem_o1, sem_g):
        wid = lax.axis_index("s") * NC + lax.axis_index("c")
        irow0 = wid * irows_pw
        obase = wid * bpw
        sem_i = (sem_i0, sem_i1)
        sem_o = (sem_o0, sem_o1)

        def load_idx(c, s):
            pltpu.async_copy(
                idx_hbm.at[pl.ds(irow0 + c * K, K)], idx_v.at[s], sem_i[s])

        def wait_idx(s):
            pltpu.make_async_copy(
                idx_hbm.at[pl.ds(irow0, K)], idx_v.at[s], sem_i[s]).wait()

        def run_gathers(s):
            copies = [
                pltpu.async_copy(
                    table_hbm.at[idx_v.at[s].at[j]],
                    rows_v.at[s].at[pl.ds(j * IW, IW)],
                    sem_g,
                )
                for j in range(K)
            ]
            for cp in copies:
                cp.wait()

        def store_rows(c, s):
            pltpu.async_copy(
                rows_v.at[s], out_hbm.at[pl.ds(obase + c * CHUNK, CHUNK)],
                sem_o[s])

        def wait_store(s):
            pltpu.make_async_copy(
                rows_v.at[s], out_hbm.at[pl.ds(obase, CHUNK)], sem_o[s]).wait()

        load_idx(0, 0)
        load_idx(1, 1)
        for s in (0, 1):
            wait_idx(s)
            run_gathers(s)
            load_idx(s + 2, s)
            store_rows(s, s)

        @pl.loop(2, nchunks - 3, step=2)
        def _(c0):
            for s in (0, 1):
                c = c0 + s
                wait_idx(s)
                wait_store(s)
                run_gathers(s)
                load_idx(c + 2, s)
                store_rows(c, s)

        for c, s, pf in ((nchunks - 3, 0, True),
                         (nchunks - 2, 1, False),
                         (nchunks - 1, 0, False)):
            wait_idx(s)
            wait_store(s)
            run_gathers(s)
            if pf:
                load_idx(c + 2, s)
            store_rows(c, s)

        wait_store(1)
        wait_store(0)

    return gather


def kernel(X, table):
    Bb, T = X.shape
    B = Bb * T
    idx = X.reshape(B // IW, IW).astype(jnp.int32)
    out = _make_gather(B)(table, idx)
    return out.reshape(Bb, T, D)
